# Initial kernel scaffold; baseline (speedup 1.0000x reference)
import jax
import jax.numpy as jnp
from jax.experimental import pallas as pl


def kernel(x):
    raise NotImplementedError("write your pallas kernel here")


# speedup vs baseline: 4.5049x; 4.5049x over previous
import functools

import jax
import jax.numpy as jnp
from jax import lax
from jax.experimental import pallas as pl
from jax.experimental.pallas import tpu as pltpu
from jax.experimental.pallas import tpu_sc as plsc

_R = 128
_N = 32768
_K = 64
_L = 16
_NV = _N // _L
_NC = 2
_NS = 16
_NW = _NC * _NS
_RPW = _R // _NW
_CAP = 256
_CV = _CAP // _L
_COMPACT_AT = _CAP - 4 * _L

import numpy as np

_NEG_INF = np.float32("-inf")


def _mono_u32(v):
    i = plsc.bitcast(v, jnp.int32)
    flip = lax.shift_right_arithmetic(i, 31) & jnp.int32(0x7FFFFFFF)
    mi = i ^ flip
    return plsc.bitcast(mi, jnp.uint32) ^ jnp.uint32(0x80000000)


def _unmono_f32(mu_vec):
    mi = plsc.bitcast(mu_vec ^ jnp.uint32(0x80000000), jnp.int32)
    flip = lax.shift_right_arithmetic(mi, 31) & jnp.int32(0x7FFFFFFF)
    return plsc.bitcast(mi ^ flip, jnp.float32)


def _body(x_hbm, out_hbm, row_v, cand_v):
    cid = lax.axis_index("c")
    sid = lax.axis_index("s")
    wid = sid * _NC + cid
    lane = jnp.arange(_L, dtype=jnp.int32)

    def count_ge(cand, off, thr, m):
        cand_s = jnp.full((_L,), cand, jnp.uint32)

        def cbody(j, acc):
            vals = cand_v[pl.ds(j * _L, _L)]
            valid = (lane + j * _L) < off
            ge = jnp.logical_and(vals >= cand_s, valid)
            return acc + jnp.where(ge, jnp.int32(1), jnp.int32(0))

        accv = lax.fori_loop(0, _CV, cbody, jnp.zeros((_L,), jnp.int32))
        cnt = jnp.sum(accv)
        return cnt + jnp.where(cand <= thr, m, jnp.int32(0))

    def bisect(off, thr, m):
        def bit_body(i, t):
            bit = (jnp.uint32(31) - i.astype(jnp.uint32))
            cand = t | lax.shift_left(jnp.uint32(1), bit)
            cnt = count_ge(cand, off, thr, m)
            return jnp.where(cnt >= _K, cand, t)

        return lax.fori_loop(0, 32, bit_body, jnp.uint32(0))

    def compact(off, thr, m):
        t = bisect(off, thr, m)
        t_s = jnp.full((_L,), t, jnp.uint32)

        def rbody(j, noff):
            vals = cand_v[pl.ds(j * _L, _L)]
            keep = jnp.logical_and(vals > t_s, (lane + j * _L) < off)
            cnt = jnp.max(plsc.all_reduce_population_count(keep))
            plsc.store_compressed(cand_v.at[pl.ds(noff, _L)], vals, mask=keep)
            return noff + cnt

        new_off = lax.fori_loop(0, _CV, rbody, jnp.int32(0))
        return new_off, t, jnp.int32(_K) - new_off

    for rr in range(_RPW):
        row = wid * _RPW + rr
        pltpu.sync_copy(x_hbm.at[row], row_v)

        def stream_body(j, carry):
            off, thr, m = carry
            v = row_v[pl.ds(j * _L, _L)]
            mu = _mono_u32(v)
            pmask = mu > jnp.full((_L,), thr, jnp.uint32)
            cnt = jnp.max(plsc.all_reduce_population_count(pmask))
            plsc.store_compressed(cand_v.at[pl.ds(off, _L)], mu, mask=pmask)
            off = off + cnt
            return lax.cond(off >= _COMPACT_AT, compact,
                            lambda o, t, mm: (o, t, mm), off, thr, m)

        off, thr, m = lax.fori_loop(
            0, _NV, stream_body,
            (jnp.int32(0), jnp.uint32(0), jnp.int32(0)))

        t_row = bisect(off, thr, m)
        tf = _unmono_f32(jnp.full((_L,), t_row, jnp.uint32))

        def mask_body(j, _):
            v = row_v[pl.ds(j * _L, _L)]
            row_v[pl.ds(j * _L, _L)] = jnp.where(v >= tf, v, _NEG_INF)
            return _

        lax.fori_loop(0, _NV, mask_body, jnp.int32(0))
        pltpu.sync_copy(row_v, out_hbm.at[row])


@jax.jit
def kernel(x):
    mesh = plsc.VectorSubcoreMesh(
        core_axis_name="c", subcore_axis_name="s",
        num_cores=_NC, num_subcores=_NS)
    run = pl.kernel(
        _body,
        out_type=jax.ShapeDtypeStruct((_R, _N), jnp.float32),
        mesh=mesh,
        scratch_types=[
            pltpu.VMEM((_N,), jnp.float32),
            pltpu.VMEM((_CAP,), jnp.uint32),
        ],
        compiler_params=pltpu.CompilerParams(needs_layout_passes=False),
    )
    return run(x)


# speedup vs baseline: 9.5080x; 2.1106x over previous
import functools

import jax
import jax.numpy as jnp
import numpy as np
from jax import lax
from jax.experimental import pallas as pl
from jax.experimental.pallas import tpu as pltpu
from jax.experimental.pallas import tpu_sc as plsc

_R = 128
_N = 32768
_K = 64
_L = 16
_NV = _N // _L
_NC = 2
_NS = 16
_NW = _NC * _NS
_RPW = _R // _NW
_G = 8
_NG = _NV // _G
_CAP = 512
_CV = _CAP // _L
_COMPACT_AT = _CAP - _G * _L

_NEG_INF = np.float32("-inf")


def _mono_u32(v):
    i = plsc.bitcast(v, jnp.int32)
    flip = lax.shift_right_arithmetic(i, 31) & jnp.int32(0x7FFFFFFF)
    mi = i ^ flip
    return plsc.bitcast(mi, jnp.uint32) ^ jnp.uint32(0x80000000)


def _unmono_f32(mu_vec):
    mi = plsc.bitcast(mu_vec ^ jnp.uint32(0x80000000), jnp.int32)
    flip = lax.shift_right_arithmetic(mi, 31) & jnp.int32(0x7FFFFFFF)
    return plsc.bitcast(mi ^ flip, jnp.float32)


def _body(x_hbm, out_hbm, row_v, cand_v, mono_v):
    cid = lax.axis_index("c")
    sid = lax.axis_index("s")
    wid = sid * _NC + cid
    lane = jnp.arange(_L, dtype=jnp.int32)

    def count_ge(cand, off, thr_m, m):
        cand_s = jnp.full((_L,), cand, jnp.uint32)
        nv = (off + _L - 1) // _L

        def cbody(j, acc):
            vals = mono_v[pl.ds(j * _L, _L)]
            valid = (lane + j * _L) < off
            ge = jnp.logical_and(vals >= cand_s, valid)
            return acc + jnp.where(ge, jnp.int32(1), jnp.int32(0))

        accv = lax.fori_loop(0, nv, cbody, jnp.zeros((_L,), jnp.int32))
        cnt = jnp.sum(accv)
        return cnt + jnp.where(cand <= thr_m, m, jnp.int32(0))

    def monoize(off):
        nv = (off + _L - 1) // _L

        def mbody(j, _):
            mono_v[pl.ds(j * _L, _L)] = _mono_u32(cand_v[pl.ds(j * _L, _L)])
            return _

        lax.fori_loop(0, nv, mbody, jnp.int32(0))

    def bisect(off, thr_m, m):
        def bit_body(i, t):
            bit = jnp.uint32(31) - i.astype(jnp.uint32)
            cand = t | lax.shift_left(jnp.uint32(1), bit)
            cnt = count_ge(cand, off, thr_m, m)
            return jnp.where(cnt >= _K, cand, t)

        return lax.fori_loop(0, 32, bit_body, jnp.uint32(0))

    def compact(off, thr_f, thr_m, m):
        monoize(off)
        t = bisect(off, thr_m, m)
        t_s = jnp.full((_L,), t, jnp.uint32)
        nv = (off + _L - 1) // _L

        def rbody(j, noff):
            mono_vals = mono_v[pl.ds(j * _L, _L)]
            vals = cand_v[pl.ds(j * _L, _L)]
            keep = jnp.logical_and(mono_vals > t_s, (lane + j * _L) < off)
            cnt = plsc.all_reduce_population_count(keep)[0]
            plsc.store_compressed(cand_v.at[pl.ds(noff, _L)], vals, mask=keep)
            return noff + cnt

        new_off = lax.fori_loop(0, nv, rbody, jnp.int32(0))
        new_thr_f = _unmono_f32(jnp.full((_L,), t, jnp.uint32))[0]
        return new_off, new_thr_f, t, jnp.int32(_K) - new_off

    for rr in range(_RPW):
        row = wid * _RPW + rr
        pltpu.sync_copy(x_hbm.at[row], row_v)

        def group_body(g, carry):
            off, thr_f, thr_m, m = carry
            thr_vec = jnp.full((_L,), thr_f, jnp.float32)
            for u in range(_G):
                v = row_v[pl.ds((g * _G + u) * _L, _L)]
                pmask = v > thr_vec
                cnt = plsc.all_reduce_population_count(pmask)[0]
                plsc.store_compressed(cand_v.at[pl.ds(off, _L)], v, mask=pmask)
                off = off + cnt
            return lax.cond(off >= _COMPACT_AT, compact,
                            lambda o, tf, tm, mm: (o, tf, tm, mm),
                            off, thr_f, thr_m, m)

        off, thr_f, thr_m, m = lax.fori_loop(
            0, _NG, group_body,
            (jnp.int32(0), jnp.float32(_NEG_INF), jnp.uint32(0),
             jnp.int32(0)))

        monoize(off)
        t_row = bisect(off, thr_m, m)
        tf = _unmono_f32(jnp.full((_L,), t_row, jnp.uint32))

        @plsc.parallel_loop(0, _NV, step=1, unroll=8)
        def mask_loop(j):
            v = row_v[pl.ds(j * _L, _L)]
            row_v[pl.ds(j * _L, _L)] = jnp.where(v >= tf, v, _NEG_INF)

        pltpu.sync_copy(row_v, out_hbm.at[row])


@jax.jit
def kernel(x):
    mesh = plsc.VectorSubcoreMesh(
        core_axis_name="c", subcore_axis_name="s",
        num_cores=_NC, num_subcores=_NS)
    run = pl.kernel(
        _body,
        out_type=jax.ShapeDtypeStruct((_R, _N), jnp.float32),
        mesh=mesh,
        scratch_types=[
            pltpu.VMEM((_N,), jnp.float32),
            pltpu.VMEM((_CAP,), jnp.float32),
            pltpu.VMEM((_CAP,), jnp.uint32),
        ],
        compiler_params=pltpu.CompilerParams(needs_layout_passes=False),
    )
    return run(x)
